# Initial kernel scaffold; baseline (speedup 1.0000x reference)
import jax
import jax.numpy as jnp
from jax.experimental import pallas as pl


def kernel(x, edge_index, W_r, expert_emb, tau, W1, W2):
    raise NotImplementedError("write your pallas kernel here")


# speedup vs baseline: 73.6256x; 73.6256x over previous
import functools

import jax
import jax.numpy as jnp
from jax import lax
from jax.experimental import pallas as pl
from jax.experimental.pallas import tpu as pltpu
from jax.experimental.pallas import tpu_sc as plsc

N = 10000
E = 320000
D = 128
NEXP = 8
NP = 10112
RPT = NP // 16
CH = 128

EP = E + N
EPAD = 331776
C1 = 81
C2 = 162
HPAD = 323584
CH1 = 79


def _mesh():
    return plsc.VectorSubcoreMesh(
        core_axis_name="c", subcore_axis_name="s",
        num_cores=2, num_subcores=16)


def _sc_hist_body(dsth, zeros1, deg_out, dst_v, ones_v, stage_v, deg_sh):
    c = lax.axis_index("c")
    s = lax.axis_index("s")
    wid = c * 16 + s
    pltpu.sync_copy(zeros1.at[pl.ds(0, RPT)], stage_v)
    pltpu.sync_copy(stage_v, deg_sh.at[pl.ds(s * RPT, RPT)])
    for j in range(CH // 16):
        ones_v[pl.ds(j * 16, 16)] = jnp.full((16,), 1.0, jnp.float32)
    pltpu.sync_copy(dsth.at[wid], dst_v)
    plsc.subcore_barrier()

    def body(b, carry):
        pltpu.sync_copy(ones_v, deg_sh.at[dst_v.at[b]], add=True)
        return carry

    lax.fori_loop(0, CH1, body, 0)
    plsc.subcore_barrier()
    off = pl.multiple_of(c * NP + s * RPT, 8)
    pltpu.sync_copy(deg_sh.at[pl.ds(s * RPT, RPT)], stage_v)
    pltpu.sync_copy(stage_v, deg_out.at[pl.ds(off, RPT)])


def _init_acc(zeros2, rows, acc, s):
    pltpu.sync_copy(zeros2.at[pl.ds(0, CH)], rows)
    for r0, sz in _ROW_CHUNKS:
        pltpu.sync_copy(rows.at[pl.ds(0, sz)],
                        acc.at[pl.ds(s * RPT + r0, sz)])


def _write_acc(acc, rows, out, c, s):
    for r0, sz in _ROW_CHUNKS:
        pltpu.sync_copy(acc.at[pl.ds(s * RPT + r0, sz)], rows.at[pl.ds(0, sz)])
        pltpu.sync_copy(rows.at[pl.ds(0, sz)],
                        out.at[c, pl.ds(s * RPT + r0, sz)])


_ROW_CHUNKS = [(r0, min(CH, RPT - r0)) for r0 in range(0, RPT, CH)]


def _sc_prop1_body(src1, dst1, xp, zeros2, s1_out, src_v, dst_v, rows, acc, sem):
    c = lax.axis_index("c")
    s = lax.axis_index("s")
    wid = c * 16 + s
    _init_acc(zeros2, rows, acc, s)
    pltpu.sync_copy(src1.at[wid], src_v)
    pltpu.sync_copy(dst1.at[wid], dst_v)
    plsc.subcore_barrier()

    def body(b, carry):
        pltpu.async_copy(xp.at[src_v.at[b]], rows, sem).wait()
        pltpu.sync_copy(rows, acc.at[dst_v.at[b]], add=True)
        return carry

    lax.fori_loop(0, C1, body, 0)
    plsc.subcore_barrier()
    _write_acc(acc, rows, s1_out, c, s)


def _sc_pidx_body(tidx2, src2, tkf, gidx_out, tidx_v, src_v, ek_v, gid_v, sem):
    c = lax.axis_index("c")
    s = lax.axis_index("s")
    wid = c * 16 + s
    pltpu.sync_copy(tidx2.at[wid], tidx_v)
    pltpu.sync_copy(src2.at[s], src_v)

    def idx_body(b, carry):
        pltpu.async_copy(tkf.at[tidx_v.at[b]], ek_v, sem).wait()
        for j in range(CH // 16):
            sv = src_v[b, pl.ds(j * 16, 16)]
            ev = ek_v[pl.ds(j * 16, 16)]
            gid_v[b, pl.ds(j * 16, 16)] = ev * NP + sv
        return carry

    lax.fori_loop(0, C2, idx_body, 0)
    pltpu.sync_copy(gid_v, gidx_out.at[wid])


_SC_CACHE = {}


def _sc_hist(dsth, zeros1):
    if "hist" not in _SC_CACHE:
        _SC_CACHE["hist"] = pl.kernel(
            _sc_hist_body,
            out_type=jax.ShapeDtypeStruct((2 * NP,), jnp.float32),
            mesh=_mesh(),
            scratch_types=[
                pltpu.VMEM((CH1, CH), jnp.int32),
                pltpu.VMEM((CH,), jnp.float32),
                pltpu.VMEM((RPT,), jnp.float32),
                pltpu.VMEM_SHARED((NP,), jnp.float32),
            ],
        )
    return _SC_CACHE["hist"](dsth, zeros1)


def _sc_prop1(src1, dst1, xp, zeros2):
    if "prop1" not in _SC_CACHE:
        _SC_CACHE["prop1"] = pl.kernel(
            _sc_prop1_body,
            out_type=jax.ShapeDtypeStruct((2, NP, D), jnp.float32),
            mesh=_mesh(),
            scratch_types=[
                pltpu.VMEM((C1, CH), jnp.int32),
                pltpu.VMEM((C1, CH), jnp.int32),
                pltpu.VMEM((CH, D), jnp.float32),
                pltpu.VMEM_SHARED((NP, D), jnp.float32),
                pltpu.SemaphoreType.DMA,
            ],
        )
    return _SC_CACHE["prop1"](src1, dst1, xp, zeros2)


def _sc_pidx(tidx2, src2, tkf):
    if "pidx" not in _SC_CACHE:
        _SC_CACHE["pidx"] = pl.kernel(
            _sc_pidx_body,
            out_type=jax.ShapeDtypeStruct((32, C2, CH), jnp.int32),
            mesh=_mesh(),
            scratch_types=[
                pltpu.VMEM((C2, CH), jnp.int32),
                pltpu.VMEM((C2, CH), jnp.int32),
                pltpu.VMEM((CH,), jnp.int32),
                pltpu.VMEM((C2, CH), jnp.int32),
                pltpu.SemaphoreType.DMA,
            ],
        )
    return _SC_CACHE["pidx"](tidx2, src2, tkf)


def _router_body(x_ref, wr_ref, emb_ref, tau_ref, fg_ref, tki_ref, gg_ref):
    xb = x_ref[...]
    wr = wr_ref[...]
    hp = jax.lax.dot_general(xb, wr, (((1,), (1,)), ((), ())),
                             preferred_element_type=jnp.float32)
    hn = hp / jnp.maximum(
        jnp.sqrt(jnp.sum(hp * hp, axis=1, keepdims=True)), 1e-12)
    ee = emb_ref[...]
    en = ee / jnp.maximum(
        jnp.sqrt(jnp.sum(ee * ee, axis=1, keepdims=True)), 1e-12)
    scores = jax.lax.dot_general(hn, en, (((1,), (1,)), ((), ())),
                                 preferred_element_type=jnp.float32)
    z = scores / tau_ref[0, 0]
    zm = jnp.max(z, axis=1, keepdims=True)
    p = jnp.exp(z - zm)
    fg = p / jnp.sum(p, axis=1, keepdims=True)
    fg_ref[...] = fg
    iota = jax.lax.broadcasted_iota(jnp.int32, fg.shape, 1)
    v1 = jnp.max(fg, axis=1, keepdims=True)
    i1 = jnp.min(jnp.where(fg == v1, iota, NEXP), axis=1, keepdims=True)
    masked = jnp.where(iota == i1, -1.0, fg)
    v2 = jnp.max(masked, axis=1, keepdims=True)
    i2 = jnp.min(jnp.where(masked == v2, iota, NEXP), axis=1, keepdims=True)
    e2 = jnp.exp(v2 - v1)
    tot = 1.0 + e2
    tki_ref[...] = jnp.concatenate([i1, i2], axis=1)
    gg_ref[...] = jnp.concatenate([1.0 / tot, e2 / tot], axis=1)


def _router_tc(x, w_r, emb, tau):
    b = 2000
    grid = (N // b,)
    return pl.pallas_call(
        _router_body,
        grid=grid,
        in_specs=[
            pl.BlockSpec((b, D), lambda i: (i, 0)),
            pl.BlockSpec((NEXP, D), lambda i: (0, 0)),
            pl.BlockSpec((NEXP, NEXP), lambda i: (0, 0)),
            pl.BlockSpec((1, 1), lambda i: (0, 0)),
        ],
        out_specs=[
            pl.BlockSpec((b, NEXP), lambda i: (i, 0)),
            pl.BlockSpec((b, 2), lambda i: (i, 0)),
            pl.BlockSpec((b, 2), lambda i: (i, 0)),
        ],
        out_shape=[
            jax.ShapeDtypeStruct((N, NEXP), jnp.float32),
            jax.ShapeDtypeStruct((N, 2), jnp.int32),
            jax.ShapeDtypeStruct((N, 2), jnp.float32),
        ],
    )(x, w_r, emb, tau)


def _scale_body(x_ref, d0_ref, d1_ref, norm_ref, xp_ref):
    deg = d0_ref[...] + d1_ref[...] + 1.0
    nr = jax.lax.rsqrt(deg)
    norm_ref[...] = nr
    xp_ref[...] = x_ref[...] * nr


def _scale_tc(xpad, deg0, deg1):
    b = 1264
    grid = (NP // b,)
    return pl.pallas_call(
        _scale_body,
        grid=grid,
        in_specs=[
            pl.BlockSpec((b, D), lambda i: (i, 0)),
            pl.BlockSpec((b, 1), lambda i: (i, 0)),
            pl.BlockSpec((b, 1), lambda i: (i, 0)),
        ],
        out_specs=[
            pl.BlockSpec((b, 1), lambda i: (i, 0)),
            pl.BlockSpec((b, D), lambda i: (i, 0)),
        ],
        out_shape=[
            jax.ShapeDtypeStruct((NP, 1), jnp.float32),
            jax.ShapeDtypeStruct((NP, D), jnp.float32),
        ],
    )(xpad, deg0, deg1)


def _experts_body(s1a_ref, s1b_ref, norm_ref, w1_ref, w2_ref, zp_ref):
    nr = norm_ref[...]
    p = (s1a_ref[...] + s1b_ref[...]) * nr
    h1 = jnp.maximum(
        jax.lax.dot_general(p, w1_ref[...], (((1,), (0,)), ((), ())),
                            preferred_element_type=jnp.float32), 0.0)
    for e in range(NEXP):
        z = jax.lax.dot_general(h1[:, e * D:(e + 1) * D], w2_ref[e],
                                (((1,), (0,)), ((), ())),
                                preferred_element_type=jnp.float32)
        zp_ref[e] = z * nr


def _experts_tc(s1a, s1b, norm, w1cat, w2):
    b = 1264
    grid = (NP // b,)
    return pl.pallas_call(
        _experts_body,
        grid=grid,
        in_specs=[
            pl.BlockSpec((b, D), lambda i: (i, 0)),
            pl.BlockSpec((b, D), lambda i: (i, 0)),
            pl.BlockSpec((b, 1), lambda i: (i, 0)),
            pl.BlockSpec((D, NEXP * D), lambda i: (0, 0)),
            pl.BlockSpec((NEXP, D, D), lambda i: (0, 0, 0)),
        ],
        out_specs=pl.BlockSpec((NEXP, b, D), lambda i: (0, i, 0)),
        out_shape=jax.ShapeDtypeStruct((NEXP, NP, D), jnp.float32),
    )(s1a, s1b, norm, w1cat, w2)


def _combine_body(u00_ref, u01_ref, u10_ref, u11_ref, norm_ref, gg_ref, out_ref):
    g0 = gg_ref[:, 0:1]
    g1 = gg_ref[:, 1:2]
    out_ref[...] = norm_ref[...] * (g0 * (u00_ref[...] + u01_ref[...]) +
                                    g1 * (u10_ref[...] + u11_ref[...]))


def _combine_tc(u00, u01, u10, u11, norm, ggpad):
    b = 1264
    grid = (NP // b,)
    return pl.pallas_call(
        _combine_body,
        grid=grid,
        in_specs=[
            pl.BlockSpec((b, D), lambda i: (i, 0)),
            pl.BlockSpec((b, D), lambda i: (i, 0)),
            pl.BlockSpec((b, D), lambda i: (i, 0)),
            pl.BlockSpec((b, D), lambda i: (i, 0)),
            pl.BlockSpec((b, 1), lambda i: (i, 0)),
            pl.BlockSpec((b, 2), lambda i: (i, 0)),
        ],
        out_specs=pl.BlockSpec((b, D), lambda i: (i, 0)),
        out_shape=jax.ShapeDtypeStruct((NP, D), jnp.float32),
    )(u00, u01, u10, u11, norm, ggpad)


def kernel(x, edge_index, W_r, expert_emb, tau, W1, W2):
    src = edge_index[0].astype(jnp.int32)
    dst = edge_index[1].astype(jnp.int32)
    selfs = jnp.arange(N, dtype=jnp.int32)

    dsth = jnp.concatenate(
        [dst, jnp.full((HPAD - E,), N, jnp.int32)]).reshape(32, CH1, CH)
    src1 = jnp.concatenate(
        [src, selfs, jnp.full((EPAD - EP,), N, jnp.int32)])
    dst1 = jnp.concatenate(
        [dst, selfs, jnp.full((EPAD - EP,), N, jnp.int32)])
    src1r = src1.reshape(32, C1, CH)
    dst1r = dst1.reshape(32, C1, CH)
    src2 = src1.reshape(16, C2, CH)
    dst2 = dst1.reshape(16, C2, CH)
    tidx2 = jnp.stack([dst2, dst2 + NP]).reshape(32, C2, CH)

    zeros1 = jnp.zeros((NP,), jnp.float32)
    zeros2 = jnp.zeros((NP, D), jnp.float32)
    xpad = jnp.pad(x, ((0, NP - N), (0, 0)))
    tau2 = jnp.asarray(tau, jnp.float32).reshape(1, 1)
    w1cat = jnp.transpose(W1, (1, 0, 2)).reshape(D, NEXP * D)

    full_gates, topk_indices, gg = _router_tc(x, W_r, expert_emb, tau2)

    deg = _sc_hist(dsth, zeros1).reshape(2, NP)
    norm, xp = _scale_tc(xpad, deg[0].reshape(NP, 1), deg[1].reshape(NP, 1))

    s1 = _sc_prop1(src1r, dst1r, xp, zeros2)
    zp = _experts_tc(s1[0], s1[1], norm, w1cat, W2)

    tkf = jnp.pad(topk_indices, ((0, NP - N), (0, 0))).T.reshape(2 * NP)
    gidx = _sc_pidx(tidx2, src2, tkf)
    zpf = zp.reshape(NEXP * NP, D)
    u0p = _sc_prop1(gidx[:16].reshape(32, C1, CH), dst1r, zpf, zeros2)
    u1p = _sc_prop1(gidx[16:].reshape(32, C1, CH), dst1r, zpf, zeros2)
    ggpad = jnp.pad(gg, ((0, NP - N), (0, 0)))
    outp = _combine_tc(u0p[0], u0p[1], u1p[0], u1p[1], norm, ggpad)

    return outp[:N], topk_indices, full_gates
